# baseline (speedup 1.0000x reference)
import functools

import jax
import jax.numpy as jnp
from jax import lax
from jax.experimental import pallas as pl
from jax.experimental.pallas import tpu as pltpu
from jax.experimental.pallas import tpu_sc as plsc

NC = 2
NS = 16
L = 16
NW = NC * NS

B = 16384
D = 64
DP = 128
BPW = B // NW
HALF = BPW // 2
CHUNK = 128
NCHUNK = HALF // CHUNK


def _bpr_body(user_hbm, item_i_hbm, item_j_hbm, euw_hbm, eiw_hbm,
              out_i_hbm, out_j_hbm,
              u_idx, i_idx, j_idx, hu, hi_, hj, u_rows, vi_rows, vj_rows,
              tile_i, tile_j, pred_i, pred_j, sem):
    wid = lax.axis_index("s") * NC + lax.axis_index("c")
    base = wid * BPW
    lane_iota = lax.iota(jnp.int32, L)

    for c in range(2 * NCHUNK):
        off = base + c * CHUNK
        pltpu.sync_copy(user_hbm.at[pl.ds(off, CHUNK)], u_idx.at[c])
        pltpu.sync_copy(item_i_hbm.at[pl.ds(off, CHUNK)], i_idx.at[c])
        pltpu.sync_copy(item_j_hbm.at[pl.ds(off, CHUNK)], j_idx.at[c])

    for c in range(2 * NCHUNK):
        for q in range(CHUNK // L):
            sl = pl.ds(q * L, L)
            hu[c, sl] = u_idx[c, sl] >> 1
            hi_[c, sl] = i_idx[c, sl] >> 1
            hj[c, sl] = j_idx[c, sl] >> 1

    for h in range(2):
        descs = []
        for c in range(NCHUNK):
            dst = pl.ds(c * CHUNK, CHUNK)
            cc = h * NCHUNK + c
            descs.append(pltpu.async_copy(
                euw_hbm.at[hu.at[cc]], u_rows.at[dst], sem))
            descs.append(pltpu.async_copy(
                eiw_hbm.at[hi_.at[cc]], vi_rows.at[dst], sem))
            descs.append(pltpu.async_copy(
                eiw_hbm.at[hj.at[cc]], vj_rows.at[dst], sem))
        for dsc in descs:
            dsc.wait()

        def group_body(g, _):
            base_r = g * L
            gc = h * NCHUNK + g // (CHUNK // L)
            go = pl.ds((g % (CHUNK // L)) * L, L)
            pu = u_idx[gc, go] & 1
            pi = i_idx[gc, go] & 1
            pj = j_idx[gc, go] & 1
            for rr in range(L):
                r = base_r + rr
                ou = pu[rr] * D
                oi = pi[rr] * D
                oj = pj[rr] * D
                acc_i = jnp.zeros((L,), jnp.float32)
                acc_j = jnp.zeros((L,), jnp.float32)
                for k in range(D // L):
                    u = u_rows[r, pl.ds(ou + k * L, L)]
                    acc_i = acc_i + u * vi_rows[r, pl.ds(oi + k * L, L)]
                    acc_j = acc_j + u * vj_rows[r, pl.ds(oj + k * L, L)]
                col = lane_iota * L + rr
                plsc.store_scatter(tile_i, [col], acc_i)
                plsc.store_scatter(tile_j, [col], acc_j)
            vec_i = tile_i[pl.ds(0, L)]
            vec_j = tile_j[pl.ds(0, L)]
            for k in range(1, L):
                vec_i = vec_i + tile_i[pl.ds(k * L, L)]
                vec_j = vec_j + tile_j[pl.ds(k * L, L)]
            pred_i[pl.ds(h * HALF + base_r, L)] = vec_i
            pred_j[pl.ds(h * HALF + base_r, L)] = vec_j
            return 0

        lax.fori_loop(0, HALF // L, group_body, 0)

    pltpu.sync_copy(pred_i, out_i_hbm.at[pl.ds(base, BPW)])
    pltpu.sync_copy(pred_j, out_j_hbm.at[pl.ds(base, BPW)])


@jax.jit
def _bpr(user, item_i, item_j, embed_user_weight, embed_item_weight):
    mesh = plsc.VectorSubcoreMesh(core_axis_name="c", subcore_axis_name="s",
                                  num_cores=NC, num_subcores=NS)
    euw = embed_user_weight.reshape(-1, DP)
    eiw = embed_item_weight.reshape(-1, DP)
    f = functools.partial(
        pl.kernel,
        out_type=(jax.ShapeDtypeStruct((B,), jnp.float32),
                  jax.ShapeDtypeStruct((B,), jnp.float32)),
        mesh=mesh,
        compiler_params=pltpu.CompilerParams(needs_layout_passes=False,
                                             use_tc_tiling_on_sc=True),
        scratch_types=[
            pltpu.VMEM((2 * NCHUNK, CHUNK), jnp.int32),
            pltpu.VMEM((2 * NCHUNK, CHUNK), jnp.int32),
            pltpu.VMEM((2 * NCHUNK, CHUNK), jnp.int32),
            pltpu.VMEM((2 * NCHUNK, CHUNK), jnp.int32),
            pltpu.VMEM((2 * NCHUNK, CHUNK), jnp.int32),
            pltpu.VMEM((2 * NCHUNK, CHUNK), jnp.int32),
            pltpu.VMEM((HALF, DP), jnp.float32),
            pltpu.VMEM((HALF, DP), jnp.float32),
            pltpu.VMEM((HALF, DP), jnp.float32),
            pltpu.VMEM((L * L,), jnp.float32),
            pltpu.VMEM((L * L,), jnp.float32),
            pltpu.VMEM((BPW,), jnp.float32),
            pltpu.VMEM((BPW,), jnp.float32),
            pltpu.SemaphoreType.DMA,
        ],
    )(_bpr_body)
    return f(user, item_i, item_j, euw, eiw)


def kernel(user, item_i, item_j, embed_user_weight, embed_item_weight):
    return _bpr(user, item_i, item_j, embed_user_weight, embed_item_weight)


# speedup vs baseline: 1.0996x; 1.0996x over previous
import functools

import jax
import jax.numpy as jnp
from jax import lax
from jax.experimental import pallas as pl
from jax.experimental.pallas import tpu as pltpu
from jax.experimental.pallas import tpu_sc as plsc

NC = 2
NS = 16
L = 16
NW = NC * NS

B = 16384
D = 64
DP = 128
BPW = B // NW
HALF = BPW // 2
CHUNK = 128
NCHUNK = HALF // CHUNK


def _bpr_body(user_hbm, item_i_hbm, item_j_hbm, euw_hbm, eiw_hbm,
              out_i_hbm, out_j_hbm,
              u_idx, i_idx, j_idx, u_rows, vi_rows, vj_rows,
              tile_i, tile_j, pred_i, pred_j, sem):
    wid = lax.axis_index("s") * NC + lax.axis_index("c")
    base = wid * BPW
    lane_iota = lax.iota(jnp.int32, L)

    for c in range(2 * NCHUNK):
        off = base + c * CHUNK
        pltpu.sync_copy(user_hbm.at[pl.ds(off, CHUNK)], u_idx.at[c])
        pltpu.sync_copy(item_i_hbm.at[pl.ds(off, CHUNK)], i_idx.at[c])
        pltpu.sync_copy(item_j_hbm.at[pl.ds(off, CHUNK)], j_idx.at[c])

    for h in range(2):
        descs = []
        for c in range(NCHUNK):
            dst = pl.ds(c * CHUNK, CHUNK)
            cc = h * NCHUNK + c
            descs.append(pltpu.async_copy(
                euw_hbm.at[u_idx.at[cc]], u_rows.at[dst], sem))
            descs.append(pltpu.async_copy(
                eiw_hbm.at[i_idx.at[cc]], vi_rows.at[dst], sem))
            descs.append(pltpu.async_copy(
                eiw_hbm.at[j_idx.at[cc]], vj_rows.at[dst], sem))
        for dsc in descs:
            dsc.wait()

        def group_body(g, _):
            base_r = g * L
            for rr in range(L):
                r = base_r + rr
                acc_i = jnp.zeros((L,), jnp.float32)
                acc_j = jnp.zeros((L,), jnp.float32)
                for k in range(D // L):
                    sl = pl.ds(k * L, L)
                    u = u_rows[r, sl]
                    acc_i = acc_i + u * vi_rows[r, sl]
                    acc_j = acc_j + u * vj_rows[r, sl]
                col = lane_iota * L + rr
                plsc.store_scatter(tile_i, [col], acc_i)
                plsc.store_scatter(tile_j, [col], acc_j)
            vec_i = tile_i[pl.ds(0, L)]
            vec_j = tile_j[pl.ds(0, L)]
            for k in range(1, L):
                vec_i = vec_i + tile_i[pl.ds(k * L, L)]
                vec_j = vec_j + tile_j[pl.ds(k * L, L)]
            pred_i[pl.ds(h * HALF + base_r, L)] = vec_i
            pred_j[pl.ds(h * HALF + base_r, L)] = vec_j
            return 0

        lax.fori_loop(0, HALF // L, group_body, 0)

    pltpu.sync_copy(pred_i, out_i_hbm.at[pl.ds(base, BPW)])
    pltpu.sync_copy(pred_j, out_j_hbm.at[pl.ds(base, BPW)])


@jax.jit
def _bpr(user, item_i, item_j, embed_user_weight, embed_item_weight):
    mesh = plsc.VectorSubcoreMesh(core_axis_name="c", subcore_axis_name="s",
                                  num_cores=NC, num_subcores=NS)
    euw = jnp.pad(embed_user_weight, ((0, 0), (0, DP - D)))
    eiw = jnp.pad(embed_item_weight, ((0, 0), (0, DP - D)))
    f = functools.partial(
        pl.kernel,
        out_type=(jax.ShapeDtypeStruct((B,), jnp.float32),
                  jax.ShapeDtypeStruct((B,), jnp.float32)),
        mesh=mesh,
        compiler_params=pltpu.CompilerParams(needs_layout_passes=False,
                                             use_tc_tiling_on_sc=True),
        scratch_types=[
            pltpu.VMEM((2 * NCHUNK, CHUNK), jnp.int32),
            pltpu.VMEM((2 * NCHUNK, CHUNK), jnp.int32),
            pltpu.VMEM((2 * NCHUNK, CHUNK), jnp.int32),
            pltpu.VMEM((HALF, DP), jnp.float32),
            pltpu.VMEM((HALF, DP), jnp.float32),
            pltpu.VMEM((HALF, DP), jnp.float32),
            pltpu.VMEM((L * L,), jnp.float32),
            pltpu.VMEM((L * L,), jnp.float32),
            pltpu.VMEM((BPW,), jnp.float32),
            pltpu.VMEM((BPW,), jnp.float32),
            pltpu.SemaphoreType.DMA,
        ],
    )(_bpr_body)
    return f(user, item_i, item_j, euw, eiw)


def kernel(user, item_i, item_j, embed_user_weight, embed_item_weight):
    return _bpr(user, item_i, item_j, embed_user_weight, embed_item_weight)
